# Initial kernel scaffold; baseline (speedup 1.0000x reference)
import jax
import jax.numpy as jnp
from jax.experimental import pallas as pl


def kernel(xz, h, edge_index, edge_attr, We1, be1, We2, be2, Wn1, bn1, Wn2, bn2):
    raise NotImplementedError("write your pallas kernel here")


# speedup vs baseline: 3.8640x; 3.8640x over previous
import functools

import jax
import jax.numpy as jnp
from jax import lax
from jax.experimental import pallas as pl
from jax.experimental.pallas import tpu as pltpu
from jax.experimental.pallas import tpu_sc as plsc

_NC = 2
_NS = 16
_NW = _NC * _NS
_CHUNK = 128


def _sc_mesh():
    return plsc.VectorSubcoreMesh(
        core_axis_name="c", subcore_axis_name="s",
        num_cores=_NC, num_subcores=_NS)


def _prep_tables(h, xzp, Wr, Wc):
    n, f = h.shape
    hid = Wr.shape[1]
    w = hid + xzp.shape[1]
    bm = 1000

    def body(h_ref, x_ref, wr_ref, wc_ref, tr_ref, tc_ref):
        hb = h_ref[...]
        xb = x_ref[...]
        tr_ref[...] = jnp.concatenate(
            [jnp.dot(hb, wr_ref[...], preferred_element_type=jnp.float32), xb],
            axis=1)
        tc_ref[...] = jnp.concatenate(
            [jnp.dot(hb, wc_ref[...], preferred_element_type=jnp.float32), xb],
            axis=1)

    return pl.pallas_call(
        body,
        grid=(n // bm,),
        in_specs=[
            pl.BlockSpec((bm, f), lambda i: (i, 0)),
            pl.BlockSpec((bm, xzp.shape[1]), lambda i: (i, 0)),
            pl.BlockSpec((f, hid), lambda i: (0, 0)),
            pl.BlockSpec((f, hid), lambda i: (0, 0)),
        ],
        out_specs=[
            pl.BlockSpec((bm, w), lambda i: (i, 0)),
            pl.BlockSpec((bm, w), lambda i: (i, 0)),
        ],
        out_shape=[jax.ShapeDtypeStruct((n, w), jnp.float32)] * 2,
    )(h, xzp, Wr, Wc)


def _sc_gather(table_r, table_c, row2d, col2d):
    n_chunks = row2d.shape[0]
    iters = (n_chunks + _NW - 1) // _NW
    w = table_r.shape[1]
    e = n_chunks * _CHUNK

    @functools.partial(
        pl.kernel, mesh=_sc_mesh(),
        compiler_params=pltpu.CompilerParams(use_tc_tiling_on_sc=False),
        out_type=[jax.ShapeDtypeStruct((e, w), jnp.float32)] * 2,
        scratch_types=[
            pltpu.VMEM((_CHUNK,), jnp.int32),
            pltpu.VMEM((_CHUNK,), jnp.int32),
            pltpu.VMEM((_CHUNK, w), jnp.float32),
            pltpu.VMEM((_CHUNK, w), jnp.float32),
            pltpu.SemaphoreType.DMA,
            pltpu.SemaphoreType.DMA,
        ],
    )
    def k(tr_hbm, tc_hbm, row_hbm, col_hbm, gr_hbm, gc_hbm,
          idx_r, idx_c, rows_r, rows_c, sem_r, sem_c):
        wid = lax.axis_index("s") * _NC + lax.axis_index("c")

        def step(i, carry):
            cid = wid * iters + i

            @pl.when(cid < n_chunks)
            def _():
                pltpu.sync_copy(row_hbm.at[cid], idx_r)
                pltpu.sync_copy(col_hbm.at[cid], idx_c)
                cpy_r = pltpu.async_copy(tr_hbm.at[idx_r], rows_r, sem_r)
                cpy_c = pltpu.async_copy(tc_hbm.at[idx_c], rows_c, sem_c)
                cpy_r.wait()
                cpy_c.wait()
                base = cid * _CHUNK
                pltpu.sync_copy(rows_r, gr_hbm.at[pl.ds(base, _CHUNK)])
                pltpu.sync_copy(rows_c, gc_hbm.at[pl.ds(base, _CHUNK)])

            return carry

        lax.fori_loop(0, iters, step, 0)

    return k(table_r, table_c, row2d, col2d)


def _edge_mlp(gr, gc, ea, wd, Wa, b1, W2, b2):
    e, w = gr.shape
    hid = W2.shape[0]
    a = ea.shape[1]
    be = 4000
    eps = 1e-6

    def body(gr_ref, gc_ref, ea_ref, wd_ref, wa_ref, b1_ref, w2_ref, b2_ref,
             out_ref):
        grb = gr_ref[...]
        gcb = gc_ref[...]
        pd = grb[:, hid:] - gcb[:, hid:]
        d2 = jnp.sum(pd * pd, axis=1, keepdims=True)
        z1 = grb[:, hid + 2:hid + 3]
        z2 = gcb[:, hid + 2:hid + 3]
        arg = 1.0 + d2 / (2.0 * z1 * z2 + eps) + eps
        dist = jnp.log(arg + jnp.sqrt((arg - 1.0) * (arg + 1.0)))
        e1 = (grb[:, :hid] + gcb[:, :hid] + dist * wd_ref[...]
              + jnp.dot(ea_ref[...], wa_ref[...],
                        preferred_element_type=jnp.float32)
              + b1_ref[...])
        u = e1 * jax.nn.sigmoid(e1)
        m = jnp.dot(u, w2_ref[...], preferred_element_type=jnp.float32) \
            + b2_ref[...]
        m = m * jax.nn.sigmoid(m)
        extra = (lax.broadcasted_iota(jnp.int32, (be, w - hid), 1) == 0)
        out_ref[...] = jnp.concatenate([m, extra.astype(jnp.float32)], axis=1)

    return pl.pallas_call(
        body,
        grid=(e // be,),
        in_specs=[
            pl.BlockSpec((be, w), lambda i: (i, 0)),
            pl.BlockSpec((be, w), lambda i: (i, 0)),
            pl.BlockSpec((be, a), lambda i: (i, 0)),
            pl.BlockSpec((1, hid), lambda i: (0, 0)),
            pl.BlockSpec((a, hid), lambda i: (0, 0)),
            pl.BlockSpec((1, hid), lambda i: (0, 0)),
            pl.BlockSpec((hid, hid), lambda i: (0, 0)),
            pl.BlockSpec((1, hid), lambda i: (0, 0)),
        ],
        out_specs=pl.BlockSpec((be, w), lambda i: (i, 0)),
        out_shape=jax.ShapeDtypeStruct((e, w), jnp.float32),
    )(gr, gc, ea, wd, Wa, b1, W2, b2)


def _sc_scatter(m_ext, row2d, zeros_slab):
    n_chunks = row2d.shape[0]
    iters = (n_chunks + _NW - 1) // _NW
    w = m_ext.shape[1]
    rows_per_tile = zeros_slab.shape[0]
    n = rows_per_tile * _NS

    @functools.partial(
        pl.kernel, mesh=_sc_mesh(),
        compiler_params=pltpu.CompilerParams(use_tc_tiling_on_sc=False),
        out_type=jax.ShapeDtypeStruct((_NC, n, w), jnp.float32),
        scratch_types=[
            pltpu.VMEM((_CHUNK,), jnp.int32),
            pltpu.VMEM((_CHUNK, w), jnp.float32),
            pltpu.VMEM_SHARED((n, w), jnp.float32),
        ],
    )
    def k(m_hbm, row_hbm, z_hbm, out_hbm, idx_v, mbuf, acc):
        c = lax.axis_index("c")
        s = lax.axis_index("s")
        wid = s * _NC + c
        r0 = s * rows_per_tile
        pltpu.sync_copy(z_hbm, acc.at[pl.ds(r0, rows_per_tile)])
        plsc.subcore_barrier()

        def step(i, carry):
            cid = wid * iters + i

            @pl.when(cid < n_chunks)
            def _():
                pltpu.sync_copy(row_hbm.at[cid], idx_v)
                pltpu.sync_copy(m_hbm.at[pl.ds(cid * _CHUNK, _CHUNK)], mbuf)
                pltpu.sync_copy(mbuf, acc.at[idx_v], add=True)

            return carry

        lax.fori_loop(0, iters, step, 0)
        plsc.subcore_barrier()
        pltpu.sync_copy(acc.at[pl.ds(r0, rows_per_tile)],
                        out_hbm.at[c, pl.ds(r0, rows_per_tile)])

    return k(m_ext, row2d, zeros_slab)


def _node_mlp(h, nom0, nom1, W1h, W1a, b1, W2, b2):
    n, f = h.shape
    hid = W1h.shape[1]
    w = nom0.shape[1]
    bn = 1000

    def body(h_ref, n0_ref, n1_ref, w1h_ref, w1a_ref, b1_ref, w2_ref, b2_ref,
             out_ref):
        hb = h_ref[...]
        ns = n0_ref[...] + n1_ref[...]
        den = jnp.maximum(ns[:, hid:hid + 1], 1.0)
        agg = ns[:, :hid] / den
        t = (jnp.dot(hb, w1h_ref[...], preferred_element_type=jnp.float32)
             + jnp.dot(agg, w1a_ref[...], preferred_element_type=jnp.float32)
             + b1_ref[...])
        t = t * jax.nn.sigmoid(t)
        out_ref[...] = hb + jnp.dot(t, w2_ref[...],
                                    preferred_element_type=jnp.float32) \
            + b2_ref[...]

    return pl.pallas_call(
        body,
        grid=(n // bn,),
        in_specs=[
            pl.BlockSpec((bn, f), lambda i: (i, 0)),
            pl.BlockSpec((bn, w), lambda i: (i, 0)),
            pl.BlockSpec((bn, w), lambda i: (i, 0)),
            pl.BlockSpec((f, hid), lambda i: (0, 0)),
            pl.BlockSpec((hid, hid), lambda i: (0, 0)),
            pl.BlockSpec((1, hid), lambda i: (0, 0)),
            pl.BlockSpec((hid, f), lambda i: (0, 0)),
            pl.BlockSpec((1, f), lambda i: (0, 0)),
        ],
        out_specs=pl.BlockSpec((bn, f), lambda i: (i, 0)),
        out_shape=jax.ShapeDtypeStruct((n, f), jnp.float32),
    )(h, nom0, nom1, W1h, W1a, b1, W2, b2)


def kernel(xz, h, edge_index, edge_attr, We1, be1, We2, be2, Wn1, bn1, Wn2,
           bn2):
    n, f = h.shape
    hid = We2.shape[0]
    row = edge_index[0]
    col = edge_index[1]
    xzp = jnp.pad(xz, ((0, 0), (0, 16 - xz.shape[1])))

    table_r, table_c = _prep_tables(h, xzp, We1[:f], We1[f:2 * f])
    row2d = row.reshape(-1, _CHUNK)
    col2d = col.reshape(-1, _CHUNK)
    gr, gc = _sc_gather(table_r, table_c, row2d, col2d)

    m_ext = _edge_mlp(gr, gc, edge_attr, We1[2 * f][None, :], We1[2 * f + 1:],
                      be1[None, :], We2, be2[None, :])

    zeros_slab = jnp.zeros((n // _NS, hid + 16), jnp.float32)
    nom_p = _sc_scatter(m_ext, row2d, zeros_slab)

    return _node_mlp(h, nom_p[0], nom_p[1], Wn1[:f], Wn1[f:], bn1[None, :],
                     Wn2, bn2[None, :])
